# Initial kernel scaffold; baseline (speedup 1.0000x reference)
import jax
import jax.numpy as jnp
from jax.experimental import pallas as pl


def kernel(residue_i, residue_j):
    raise NotImplementedError("write your pallas kernel here")


# speedup vs baseline: 24.1645x; 24.1645x over previous
import jax
import jax.numpy as jnp
from jax import lax
from jax.experimental import pallas as pl
from jax.experimental.pallas import tpu as pltpu
from jax.experimental.pallas import tpu_sc as plsc

_ROWS = 16
_N = 1048576
_HALF = _N // 2
_CH = 32768
_NCHUNK = _HALF // _CH
_U = 16
_L = 16


def _sc_count_body(ri_hbm, rj_hbm, out_hbm, bi, bj, stage):
    r = lax.axis_index("s")
    h = lax.axis_index("c")
    col0 = h * _HALF

    def chunk_body(c, accs):
        off = col0 + c * _CH
        pltpu.sync_copy(ri_hbm.at[r, pl.ds(off, _CH)], bi)
        pltpu.sync_copy(rj_hbm.at[r, pl.ds(off, _CH)], bj)

        one = jnp.ones((_L,), jnp.int32)
        zero = jnp.zeros((_L,), jnp.int32)

        def inner(k, accs2):
            ai, aj, asum = accs2
            base = k * (_U * _L)
            for u in range(_U):
                o = base + u * _L
                xi = bi[pl.ds(o, _L)]
                xj = bj[pl.ds(o, _L)]
                s = xi + xj
                ai = ai + jnp.where(xi > 0, one, zero)
                aj = aj + jnp.where(xj > 0, one, zero)
                asum = asum + jnp.where(s > 0, one, zero)
            return ai, aj, asum

        return lax.fori_loop(0, _CH // (_U * _L), inner, accs)

    z = jnp.zeros((_L,), jnp.int32)
    ai, aj, asum = lax.fori_loop(0, _NCHUNK, chunk_body, (z, z, z))

    stage[pl.ds(0, _L)] = ai
    stage[pl.ds(_L, _L)] = aj
    stage[pl.ds(2 * _L, _L)] = asum
    pltpu.sync_copy(stage, out_hbm.at[h, r])


_SC_COUNTS_CACHE = []


def _sc_counts(ri, rj):
    if not _SC_COUNTS_CACHE:
        _SC_COUNTS_CACHE.append(pl.kernel(
            _sc_count_body,
            out_type=jax.ShapeDtypeStruct((2, _ROWS, 3 * _L), jnp.int32),
            mesh=plsc.VectorSubcoreMesh(
                core_axis_name="c", subcore_axis_name="s",
                num_cores=2, num_subcores=16),
            scratch_types=[
                pltpu.VMEM((_CH,), jnp.float32),
                pltpu.VMEM((_CH,), jnp.float32),
                pltpu.VMEM((3 * _L,), jnp.int32),
            ],
        ))
    return _SC_COUNTS_CACHE[0](ri, rj)


def _entropy_body(cnt_ref, out_ref):
    c = (cnt_ref[0] + cnt_ref[1]).astype(jnp.float32)
    ci = jnp.sum(c[:, 0:_L], axis=1, keepdims=True)
    cj = jnp.sum(c[:, _L:2 * _L], axis=1, keepdims=True)
    cs = jnp.sum(c[:, 2 * _L:3 * _L], axis=1, keepdims=True)

    def H(cnt):
        denom = jnp.float32(_N) + jnp.float32(1e-8)
        p1 = cnt / denom
        p0 = (jnp.float32(_N) - cnt) / denom
        log2e = jnp.float32(1.4426950408889634)
        t1 = jnp.where(p1 > 0, p1 * (jnp.log(p1 + 1e-10) * log2e), 0.0)
        t0 = jnp.where(p0 > 0, p0 * (jnp.log(p0 + 1e-10) * log2e), 0.0)
        return -(t0 + t1)

    out_ref[...] = H(cs) - H(ci) - H(cj)


def _entropy(cnts):
    return pl.pallas_call(
        _entropy_body,
        out_shape=jax.ShapeDtypeStruct((_ROWS, 1), jnp.float32),
    )(cnts)


def kernel(residue_i, residue_j):
    cnts = _sc_counts(residue_i, residue_j)
    return _entropy(cnts).reshape(_ROWS)


# speedup vs baseline: 36.6836x; 1.5181x over previous
import jax
import jax.numpy as jnp
from jax import lax
from jax.experimental import pallas as pl
from jax.experimental.pallas import tpu as pltpu
from jax.experimental.pallas import tpu_sc as plsc

_ROWS = 16
_N = 1048576
_HALF = _N // 2
_CH = 16384
_NCHUNK = _HALF // _CH
_U = 16
_L = 16


def _sc_count_body(ri_hbm, rj_hbm, out_hbm,
                   bi0, bi1, bj0, bj1, stage, si0, si1, sj0, sj1):
    r = lax.axis_index("s")
    h = lax.axis_index("c")
    col0 = h * _HALF
    bi = (bi0, bi1)
    bj = (bj0, bj1)
    si = (si0, si1)
    sj = (sj0, sj1)

    def start(c, b):
        off = col0 + c * _CH
        pltpu.make_async_copy(ri_hbm.at[r, pl.ds(off, _CH)], bi[b], si[b]).start()
        pltpu.make_async_copy(rj_hbm.at[r, pl.ds(off, _CH)], bj[b], sj[b]).start()

    def wait(c, b):
        off = col0 + c * _CH
        pltpu.make_async_copy(ri_hbm.at[r, pl.ds(off, _CH)], bi[b], si[b]).wait()
        pltpu.make_async_copy(rj_hbm.at[r, pl.ds(off, _CH)], bj[b], sj[b]).wait()

    one = jnp.ones((_L,), jnp.int32)
    zero = jnp.zeros((_L,), jnp.int32)

    def compute(bib, bjb, accs):
        def inner(k, accs2):
            ai, aj, asum = accs2
            base = k * (_U * _L)
            for u in range(_U):
                o = base + u * _L
                xi = bib[pl.ds(o, _L)]
                xj = bjb[pl.ds(o, _L)]
                s = xi + xj
                ai = ai + jnp.where(xi > 0, one, zero)
                aj = aj + jnp.where(xj > 0, one, zero)
                asum = asum + jnp.where(s > 0, one, zero)
            return ai, aj, asum

        return lax.fori_loop(0, _CH // (_U * _L), inner, accs)

    def step(c, b, accs):
        wait(c, b)

        @pl.when(c + 1 < _NCHUNK)
        def _():
            start(c + 1, 1 - b)

        return compute(bi[b], bj[b], accs)

    start(0, 0)

    def pair(c2, accs):
        c = 2 * c2
        accs = step(c, 0, accs)
        return step(c + 1, 1, accs)

    z = jnp.zeros((_L,), jnp.int32)
    ai, aj, asum = lax.fori_loop(0, _NCHUNK // 2, pair, (z, z, z))

    stage[pl.ds(0, _L)] = ai
    stage[pl.ds(_L, _L)] = aj
    stage[pl.ds(2 * _L, _L)] = asum
    pltpu.sync_copy(stage, out_hbm.at[h, r])


_SC_COUNTS_CACHE = []


def _sc_counts(ri, rj):
    if not _SC_COUNTS_CACHE:
        _SC_COUNTS_CACHE.append(pl.kernel(
            _sc_count_body,
            out_type=jax.ShapeDtypeStruct((2, _ROWS, 3 * _L), jnp.int32),
            mesh=plsc.VectorSubcoreMesh(
                core_axis_name="c", subcore_axis_name="s",
                num_cores=2, num_subcores=16),
            scratch_types=[
                pltpu.VMEM((_CH,), jnp.float32),
                pltpu.VMEM((_CH,), jnp.float32),
                pltpu.VMEM((_CH,), jnp.float32),
                pltpu.VMEM((_CH,), jnp.float32),
                pltpu.VMEM((3 * _L,), jnp.int32),
                pltpu.SemaphoreType.DMA,
                pltpu.SemaphoreType.DMA,
                pltpu.SemaphoreType.DMA,
                pltpu.SemaphoreType.DMA,
            ],
        ))
    return _SC_COUNTS_CACHE[0](ri, rj)


def _entropy_body(cnt_ref, out_ref):
    c = (cnt_ref[0] + cnt_ref[1]).astype(jnp.float32)
    ci = jnp.sum(c[:, 0:_L], axis=1, keepdims=True)
    cj = jnp.sum(c[:, _L:2 * _L], axis=1, keepdims=True)
    cs = jnp.sum(c[:, 2 * _L:3 * _L], axis=1, keepdims=True)

    def H(cnt):
        denom = jnp.float32(_N) + jnp.float32(1e-8)
        p1 = cnt / denom
        p0 = (jnp.float32(_N) - cnt) / denom
        log2e = jnp.float32(1.4426950408889634)
        t1 = jnp.where(p1 > 0, p1 * (jnp.log(p1 + 1e-10) * log2e), 0.0)
        t0 = jnp.where(p0 > 0, p0 * (jnp.log(p0 + 1e-10) * log2e), 0.0)
        return -(t0 + t1)

    out_ref[...] = H(cs) - H(ci) - H(cj)


def _entropy(cnts):
    return pl.pallas_call(
        _entropy_body,
        out_shape=jax.ShapeDtypeStruct((_ROWS, 1), jnp.float32),
    )(cnts)


def kernel(residue_i, residue_j):
    cnts = _sc_counts(residue_i, residue_j)
    return _entropy(cnts).reshape(_ROWS)


# speedup vs baseline: 36.8614x; 1.0048x over previous
import jax
import jax.numpy as jnp
from jax import lax
from jax.experimental import pallas as pl
from jax.experimental.pallas import tpu as pltpu
from jax.experimental.pallas import tpu_sc as plsc

_ROWS = 16
_N = 1048576
_HALF = _N // 2
_CH = 16384
_NCHUNK = _HALF // _CH
_U = 16
_L = 16


def _sc_count_body(ri_hbm, rj_hbm, out_hbm,
                   bi0, bi1, bj0, bj1, stage, si0, si1, sj0, sj1):
    r = lax.axis_index("s")
    h = lax.axis_index("c")
    col0 = h * _HALF
    bi = (bi0, bi1)
    bj = (bj0, bj1)
    si = (si0, si1)
    sj = (sj0, sj1)

    def start(c, b):
        off = col0 + c * _CH
        pltpu.make_async_copy(ri_hbm.at[r, pl.ds(off, _CH)], bi[b], si[b]).start()
        pltpu.make_async_copy(rj_hbm.at[r, pl.ds(off, _CH)], bj[b], sj[b]).start()

    def wait(c, b):
        off = col0 + c * _CH
        pltpu.make_async_copy(ri_hbm.at[r, pl.ds(off, _CH)], bi[b], si[b]).wait()
        pltpu.make_async_copy(rj_hbm.at[r, pl.ds(off, _CH)], bj[b], sj[b]).wait()

    one = jnp.ones((_L,), jnp.int32)
    zero = jnp.zeros((_L,), jnp.int32)

    def compute(bib, bjb, accs):
        def inner(k, accs2):
            acc = list(accs2)
            base = k * (_U * _L)
            for u in range(_U):
                o = base + u * _L
                p = u % 2
                xi = bib[pl.ds(o, _L)]
                xj = bjb[pl.ds(o, _L)]
                s = xi + xj
                acc[p] = acc[p] + jnp.where(xi > 0, one, zero)
                acc[2 + p] = acc[2 + p] + jnp.where(xj > 0, one, zero)
                acc[4 + p] = acc[4 + p] + jnp.where(s > 0, one, zero)
            return tuple(acc)

        return lax.fori_loop(0, _CH // (_U * _L), inner, accs)

    def step(c, b, accs):
        wait(c, b)

        @pl.when(c + 1 < _NCHUNK)
        def _():
            start(c + 1, 1 - b)

        return compute(bi[b], bj[b], accs)

    start(0, 0)

    def pair(c2, accs):
        c = 2 * c2
        accs = step(c, 0, accs)
        return step(c + 1, 1, accs)

    z = jnp.zeros((_L,), jnp.int32)
    accs = lax.fori_loop(0, _NCHUNK // 2, pair, (z,) * 6)

    stage[pl.ds(0, _L)] = accs[0] + accs[1]
    stage[pl.ds(_L, _L)] = accs[2] + accs[3]
    stage[pl.ds(2 * _L, _L)] = accs[4] + accs[5]
    pltpu.sync_copy(stage, out_hbm.at[h, r])


_SC_COUNTS_CACHE = []


def _sc_counts(ri, rj):
    if not _SC_COUNTS_CACHE:
        _SC_COUNTS_CACHE.append(pl.kernel(
            _sc_count_body,
            out_type=jax.ShapeDtypeStruct((2, _ROWS, 3 * _L), jnp.int32),
            mesh=plsc.VectorSubcoreMesh(
                core_axis_name="c", subcore_axis_name="s",
                num_cores=2, num_subcores=16),
            scratch_types=[
                pltpu.VMEM((_CH,), jnp.float32),
                pltpu.VMEM((_CH,), jnp.float32),
                pltpu.VMEM((_CH,), jnp.float32),
                pltpu.VMEM((_CH,), jnp.float32),
                pltpu.VMEM((3 * _L,), jnp.int32),
                pltpu.SemaphoreType.DMA,
                pltpu.SemaphoreType.DMA,
                pltpu.SemaphoreType.DMA,
                pltpu.SemaphoreType.DMA,
            ],
        ))
    return _SC_COUNTS_CACHE[0](ri, rj)


def _entropy_body(cnt_ref, out_ref):
    c = (cnt_ref[0] + cnt_ref[1]).astype(jnp.float32)
    ci = jnp.sum(c[:, 0:_L], axis=1, keepdims=True)
    cj = jnp.sum(c[:, _L:2 * _L], axis=1, keepdims=True)
    cs = jnp.sum(c[:, 2 * _L:3 * _L], axis=1, keepdims=True)

    def H(cnt):
        denom = jnp.float32(_N) + jnp.float32(1e-8)
        p1 = cnt / denom
        p0 = (jnp.float32(_N) - cnt) / denom
        log2e = jnp.float32(1.4426950408889634)
        t1 = jnp.where(p1 > 0, p1 * (jnp.log(p1 + 1e-10) * log2e), 0.0)
        t0 = jnp.where(p0 > 0, p0 * (jnp.log(p0 + 1e-10) * log2e), 0.0)
        return -(t0 + t1)

    out_ref[...] = H(cs) - H(ci) - H(cj)


def _entropy(cnts):
    return pl.pallas_call(
        _entropy_body,
        out_shape=jax.ShapeDtypeStruct((_ROWS, 1), jnp.float32),
    )(cnts)


def kernel(residue_i, residue_j):
    cnts = _sc_counts(residue_i, residue_j)
    return _entropy(cnts).reshape(_ROWS)
